# Initial kernel scaffold; baseline (speedup 1.0000x reference)
import functools

import jax
import jax.numpy as jnp
from jax.experimental import pallas as pl

UP_RATIO = 10
K = 5
STD_FACTOR = 3.0

_SELF_TQ = 256
_MAIN_TQ = 256


def _self_knn_body(q_ref, pt_ref, out_ref, *, n):
    tq = q_ref.shape[1]
    qx = q_ref[0, :, 0:1]
    qy = q_ref[0, :, 1:2]
    qz = q_ref[0, :, 2:3]
    px = pt_ref[0, 0:1, :]
    py = pt_ref[0, 1:2, :]
    pz = pt_ref[0, 2:3, :]
    dx = qx - px
    dy = qy - py
    dz = qz - pz
    d2 = dx * dx + dy * dy + dz * dz
    lane = jax.lax.broadcasted_iota(jnp.int32, (tq, n), 1)
    row = jax.lax.broadcasted_iota(jnp.int32, (tq, n), 0)
    gidx = row + pl.program_id(1) * tq
    d2 = jnp.where(lane == gidx, jnp.inf, d2)
    out_ref[0] = jnp.min(d2, axis=1, keepdims=True)


def _main_body(q_ref, ptt_ref, pts_ref, pmt_ref, pms_ref, out_ref, *, n):
    tq = q_ref.shape[1]
    qx = q_ref[0, :, 0:1]
    qy = q_ref[0, :, 1:2]
    qz = q_ref[0, :, 2:3]
    lane = jax.lax.broadcasted_iota(jnp.int32, (tq, n), 1)

    def dist2(pt_ref):
        dx = qx - pt_ref[0, 0:1, :]
        dy = qy - pt_ref[0, 1:2, :]
        dz = qz - pt_ref[0, 2:3, :]
        return dx * dx + dy * dy + dz * dz

    def top5(d2):
        mins = []
        hots = []
        for _ in range(K):
            m = jnp.min(d2, axis=1, keepdims=True)
            cand = jnp.where(d2 <= m, lane, n)
            istar = jnp.min(cand, axis=1, keepdims=True)
            onehot = lane == istar
            mins.append(m)
            hots.append(onehot)
            d2 = jnp.where(onehot, jnp.inf, d2)
        return mins, hots

    d2t = dist2(ptt_ref)
    mt, ht = top5(d2t)
    inv = [1.0 / (m + 1e-8) for m in mt]
    norm = inv[0] + inv[1] + inv[2] + inv[3] + inv[4]

    udf_t = jnp.zeros((tq, 1), jnp.float32)
    u_t = jnp.zeros((tq, n), jnp.float32)
    for k in range(K):
        udf_t = udf_t + jnp.sqrt(mt[k] + 1e-10) * inv[k]
        u_t = u_t + jnp.where(ht[k], inv[k], 0.0)

    d2s = dist2(pts_ref)
    ms, hs = top5(d2s)
    udf_s = jnp.zeros((tq, 1), jnp.float32)
    u_s = jnp.zeros((tq, n), jnp.float32)
    for k in range(K):
        udf_s = udf_s + jnp.sqrt(ms[k] + 1e-10) * inv[k]
        u_s = u_s + jnp.where(hs[k], inv[k], 0.0)

    wp_t = jnp.dot(u_t, pmt_ref[0], preferred_element_type=jnp.float32)
    wp_s = jnp.dot(u_s, pms_ref[0], preferred_element_type=jnp.float32)
    gd = jnp.sum(jnp.abs(wp_t - wp_s), axis=1, keepdims=True) / norm
    err = jnp.abs(udf_t - udf_s) / norm + gd
    out_ref[0] = err


@jax.jit
def kernel(src, tgt):
    b, n, _ = tgt.shape
    nq = n * UP_RATIO + src.shape[1]

    tgt_t = jnp.swapaxes(tgt, 1, 2)
    src_t = jnp.swapaxes(src, 1, 2)

    self_d2 = pl.pallas_call(
        functools.partial(_self_knn_body, n=n),
        grid=(b, n // _SELF_TQ),
        in_specs=[
            pl.BlockSpec((1, _SELF_TQ, 3), lambda i, j: (i, j, 0)),
            pl.BlockSpec((1, 3, n), lambda i, j: (i, 0, 0)),
        ],
        out_specs=pl.BlockSpec((1, _SELF_TQ, 1), lambda i, j: (i, j, 0)),
        out_shape=jax.ShapeDtypeStruct((b, n, 1), jnp.float32),
    )(tgt, tgt_t)

    std = jnp.sqrt(self_d2 + 1e-10) * STD_FACTOR
    noise = jax.random.normal(
        jax.random.key(42), (b, n, UP_RATIO, 3), dtype=jnp.float32
    ) * std[..., None]
    query = (tgt[:, :, None, :] + noise).reshape(b, -1, 3)
    query = jnp.concatenate([query, src], axis=1)

    pad = jnp.zeros((b, n, 5), jnp.float32)
    tgt_pad = jnp.concatenate([tgt, pad], axis=2)
    src_pad = jnp.concatenate([src, pad], axis=2)

    err = pl.pallas_call(
        functools.partial(_main_body, n=n),
        grid=(b, nq // _MAIN_TQ),
        in_specs=[
            pl.BlockSpec((1, _MAIN_TQ, 3), lambda i, j: (i, j, 0)),
            pl.BlockSpec((1, 3, n), lambda i, j: (i, 0, 0)),
            pl.BlockSpec((1, 3, n), lambda i, j: (i, 0, 0)),
            pl.BlockSpec((1, n, 8), lambda i, j: (i, 0, 0)),
            pl.BlockSpec((1, n, 8), lambda i, j: (i, 0, 0)),
        ],
        out_specs=pl.BlockSpec((1, _MAIN_TQ, 1), lambda i, j: (i, j, 0)),
        out_shape=jax.ShapeDtypeStruct((b, nq, 1), jnp.float32),
    )(query, tgt_t, src_t, tgt_pad, src_pad)

    return jnp.sum(err) / b / nq


# speedup vs baseline: 15.3032x; 15.3032x over previous
import functools

import jax
import jax.numpy as jnp
from jax.experimental import pallas as pl

UP_RATIO = 10
K = 5
STD_FACTOR = 3.0

_SELF_TQ = 256
_MAIN_TQ = 256


def _coords(ref):
    return ref[0, :, 0:1], ref[0, :, 1:2], ref[0, :, 2:3]


def _rows(ref):
    return ref[0, 0:1, :], ref[0, 1:2, :], ref[0, 2:3, :]


def _bf16(x):
    return x.astype(jnp.bfloat16).astype(jnp.float32)


def _sel_and_exact_d2(q, p):
    qx, qy, qz = q
    px, py, pz = p
    q2 = qx * qx + qy * qy + qz * qz
    p2 = px * px + py * py + pz * pz
    qp = _bf16(qx) * _bf16(px) + _bf16(qy) * _bf16(py) + _bf16(qz) * _bf16(pz)
    d2_sel = (q2 + p2) - 2.0 * qp
    dx = qx - px
    dy = qy - py
    dz = qz - pz
    d2_exact = dx * dx + dy * dy + dz * dz
    return d2_sel, d2_exact


def _self_knn_body(q_ref, pt_ref, out_ref, *, n):
    tq = q_ref.shape[1]
    q = _coords(q_ref)
    p = _rows(pt_ref)
    d2_sel, d2_exact = _sel_and_exact_d2(q, p)
    lane = jax.lax.broadcasted_iota(jnp.int32, (tq, n), 1)
    for rank in range(2):
        m = jnp.min(d2_sel, axis=1, keepdims=True)
        cand = jnp.where(d2_sel <= m, lane, n)
        istar = jnp.min(cand, axis=1, keepdims=True)
        onehot = lane == istar
        if rank == 1:
            out_ref[0] = jnp.sum(jnp.where(onehot, d2_exact, 0.0),
                                 axis=1, keepdims=True)
        d2_sel = jnp.where(onehot, jnp.inf, d2_sel)


def _main_body(q_ref, ptt_ref, pts_ref, pmt_ref, pms_ref, out_ref, *, n):
    tq = q_ref.shape[1]
    q = _coords(q_ref)
    lane = jax.lax.broadcasted_iota(jnp.int32, (tq, n), 1)

    def top5(pt_ref):
        d2_sel, d2_exact = _sel_and_exact_d2(q, _rows(pt_ref))
        dists = []
        hots = []
        for _ in range(K):
            m = jnp.min(d2_sel, axis=1, keepdims=True)
            cand = jnp.where(d2_sel <= m, lane, n)
            istar = jnp.min(cand, axis=1, keepdims=True)
            onehot = lane == istar
            dists.append(jnp.sum(jnp.where(onehot, d2_exact, 0.0),
                                 axis=1, keepdims=True))
            hots.append(onehot)
            d2_sel = jnp.where(onehot, jnp.inf, d2_sel)
        return dists, hots

    mt, ht = top5(ptt_ref)
    inv = [1.0 / (m + 1e-8) for m in mt]
    norm = inv[0] + inv[1] + inv[2] + inv[3] + inv[4]

    udf_t = jnp.zeros((tq, 1), jnp.float32)
    u_t = jnp.zeros((tq, n), jnp.float32)
    for k in range(K):
        udf_t = udf_t + jnp.sqrt(mt[k] + 1e-10) * inv[k]
        u_t = u_t + jnp.where(ht[k], inv[k], 0.0)

    ms, hs = top5(pts_ref)
    udf_s = jnp.zeros((tq, 1), jnp.float32)
    u_s = jnp.zeros((tq, n), jnp.float32)
    for k in range(K):
        udf_s = udf_s + jnp.sqrt(ms[k] + 1e-10) * inv[k]
        u_s = u_s + jnp.where(hs[k], inv[k], 0.0)

    wp_t = jnp.dot(u_t, pmt_ref[0], preferred_element_type=jnp.float32,
                   precision=jax.lax.Precision.HIGHEST)
    wp_s = jnp.dot(u_s, pms_ref[0], preferred_element_type=jnp.float32,
                   precision=jax.lax.Precision.HIGHEST)
    gd = jnp.sum(jnp.abs(wp_t - wp_s), axis=1, keepdims=True) / norm
    err = jnp.abs(udf_t - udf_s) / norm + gd
    out_ref[0] = err


@jax.jit
def kernel(src, tgt):
    b, n, _ = tgt.shape
    nq = n * UP_RATIO + src.shape[1]

    tgt_t = jnp.swapaxes(tgt, 1, 2)
    src_t = jnp.swapaxes(src, 1, 2)

    self_d2 = pl.pallas_call(
        functools.partial(_self_knn_body, n=n),
        grid=(b, n // _SELF_TQ),
        in_specs=[
            pl.BlockSpec((1, _SELF_TQ, 3), lambda i, j: (i, j, 0)),
            pl.BlockSpec((1, 3, n), lambda i, j: (i, 0, 0)),
        ],
        out_specs=pl.BlockSpec((1, _SELF_TQ, 1), lambda i, j: (i, j, 0)),
        out_shape=jax.ShapeDtypeStruct((b, n, 1), jnp.float32),
    )(tgt, tgt_t)

    std = jnp.sqrt(self_d2 + 1e-10) * STD_FACTOR
    noise = jax.random.normal(
        jax.random.key(42), (b, n, UP_RATIO, 3), dtype=jnp.float32
    ) * std[..., None]
    query = (tgt[:, :, None, :] + noise).reshape(b, -1, 3)
    query = jnp.concatenate([query, src], axis=1)

    pad = jnp.zeros((b, n, 5), jnp.float32)
    tgt_pad = jnp.concatenate([tgt, pad], axis=2)
    src_pad = jnp.concatenate([src, pad], axis=2)

    err = pl.pallas_call(
        functools.partial(_main_body, n=n),
        grid=(b, nq // _MAIN_TQ),
        in_specs=[
            pl.BlockSpec((1, _MAIN_TQ, 3), lambda i, j: (i, j, 0)),
            pl.BlockSpec((1, 3, n), lambda i, j: (i, 0, 0)),
            pl.BlockSpec((1, 3, n), lambda i, j: (i, 0, 0)),
            pl.BlockSpec((1, n, 8), lambda i, j: (i, 0, 0)),
            pl.BlockSpec((1, n, 8), lambda i, j: (i, 0, 0)),
        ],
        out_specs=pl.BlockSpec((1, _MAIN_TQ, 1), lambda i, j: (i, j, 0)),
        out_shape=jax.ShapeDtypeStruct((b, nq, 1), jnp.float32),
    )(query, tgt_t, src_t, tgt_pad, src_pad)

    return jnp.sum(err) / b / nq


# speedup vs baseline: 18.8927x; 1.2346x over previous
import functools

import jax
import jax.numpy as jnp
from jax.experimental import pallas as pl

UP_RATIO = 10
K = 5
STD_FACTOR = 3.0

_SELF_TQ = 256
_MAIN_TQ = 256


def _coords(ref):
    return ref[0, :, 0:1], ref[0, :, 1:2], ref[0, :, 2:3]


def _rows(ref):
    return ref[0, 0:1, :], ref[0, 1:2, :], ref[0, 2:3, :]


def _bf16(x):
    return x.astype(jnp.bfloat16).astype(jnp.float32)


def _sel_and_exact_d2(q, p):
    qx, qy, qz = q
    px, py, pz = p
    q2 = qx * qx + qy * qy + qz * qz
    p2 = px * px + py * py + pz * pz
    qp = _bf16(qx) * _bf16(px) + _bf16(qy) * _bf16(py) + _bf16(qz) * _bf16(pz)
    d2_sel = (q2 + p2) - 2.0 * qp
    dx = qx - px
    dy = qy - py
    dz = qz - pz
    d2_exact = dx * dx + dy * dy + dz * dz
    return d2_sel, d2_exact


def _self_knn_body(q_ref, pt_ref, out_ref, *, n):
    tq = q_ref.shape[1]
    q = _coords(q_ref)
    p = _rows(pt_ref)
    d2_sel, d2_exact = _sel_and_exact_d2(q, p)
    for rank in range(2):
        m = jnp.min(d2_sel, axis=1, keepdims=True)
        onehot = d2_sel <= m
        if rank == 1:
            out_ref[0] = jnp.sum(jnp.where(onehot, d2_exact, 0.0),
                                 axis=1, keepdims=True)
        else:
            d2_sel = jnp.where(onehot, jnp.inf, d2_sel)


def _main_body(q_ref, ptt_ref, pts_ref, pmt_ref, pms_ref, out_ref, *, n):
    tq = q_ref.shape[1]
    q = _coords(q_ref)

    def top5(pt_ref):
        d2_sel, d2_exact = _sel_and_exact_d2(q, _rows(pt_ref))
        dists = []
        hots = []
        for k in range(K):
            m = jnp.min(d2_sel, axis=1, keepdims=True)
            onehot = d2_sel <= m
            dists.append(jnp.sum(jnp.where(onehot, d2_exact, 0.0),
                                 axis=1, keepdims=True))
            hots.append(onehot)
            if k < K - 1:
                d2_sel = jnp.where(onehot, jnp.inf, d2_sel)
        return dists, hots

    mt, ht = top5(ptt_ref)
    inv = [1.0 / (m + 1e-8) for m in mt]
    norm = inv[0] + inv[1] + inv[2] + inv[3] + inv[4]

    udf_t = jnp.zeros((tq, 1), jnp.float32)
    u_t = jnp.zeros((tq, n), jnp.float32)
    for k in range(K):
        udf_t = udf_t + jnp.sqrt(mt[k] + 1e-10) * inv[k]
        u_t = u_t + jnp.where(ht[k], inv[k], 0.0)

    ms, hs = top5(pts_ref)
    udf_s = jnp.zeros((tq, 1), jnp.float32)
    u_s = jnp.zeros((tq, n), jnp.float32)
    for k in range(K):
        udf_s = udf_s + jnp.sqrt(ms[k] + 1e-10) * inv[k]
        u_s = u_s + jnp.where(hs[k], inv[k], 0.0)

    wp_t = jnp.dot(u_t, pmt_ref[0], preferred_element_type=jnp.float32,
                   precision=jax.lax.Precision.HIGHEST)
    wp_s = jnp.dot(u_s, pms_ref[0], preferred_element_type=jnp.float32,
                   precision=jax.lax.Precision.HIGHEST)
    gd = jnp.sum(jnp.abs(wp_t - wp_s), axis=1, keepdims=True) / norm
    err = jnp.abs(udf_t - udf_s) / norm + gd
    out_ref[0] = err


@jax.jit
def kernel(src, tgt):
    b, n, _ = tgt.shape
    nq = n * UP_RATIO + src.shape[1]

    tgt_t = jnp.swapaxes(tgt, 1, 2)
    src_t = jnp.swapaxes(src, 1, 2)

    self_d2 = pl.pallas_call(
        functools.partial(_self_knn_body, n=n),
        grid=(b, n // _SELF_TQ),
        in_specs=[
            pl.BlockSpec((1, _SELF_TQ, 3), lambda i, j: (i, j, 0)),
            pl.BlockSpec((1, 3, n), lambda i, j: (i, 0, 0)),
        ],
        out_specs=pl.BlockSpec((1, _SELF_TQ, 1), lambda i, j: (i, j, 0)),
        out_shape=jax.ShapeDtypeStruct((b, n, 1), jnp.float32),
    )(tgt, tgt_t)

    std = jnp.sqrt(self_d2 + 1e-10) * STD_FACTOR
    noise = jax.random.normal(
        jax.random.key(42), (b, n, UP_RATIO, 3), dtype=jnp.float32
    ) * std[..., None]
    query = (tgt[:, :, None, :] + noise).reshape(b, -1, 3)
    query = jnp.concatenate([query, src], axis=1)

    pad = jnp.zeros((b, n, 5), jnp.float32)
    tgt_pad = jnp.concatenate([tgt, pad], axis=2)
    src_pad = jnp.concatenate([src, pad], axis=2)

    err = pl.pallas_call(
        functools.partial(_main_body, n=n),
        grid=(b, nq // _MAIN_TQ),
        in_specs=[
            pl.BlockSpec((1, _MAIN_TQ, 3), lambda i, j: (i, j, 0)),
            pl.BlockSpec((1, 3, n), lambda i, j: (i, 0, 0)),
            pl.BlockSpec((1, 3, n), lambda i, j: (i, 0, 0)),
            pl.BlockSpec((1, n, 8), lambda i, j: (i, 0, 0)),
            pl.BlockSpec((1, n, 8), lambda i, j: (i, 0, 0)),
        ],
        out_specs=pl.BlockSpec((1, _MAIN_TQ, 1), lambda i, j: (i, j, 0)),
        out_shape=jax.ShapeDtypeStruct((b, nq, 1), jnp.float32),
    )(query, tgt_t, src_t, tgt_pad, src_pad)

    return jnp.sum(err) / b / nq


# speedup vs baseline: 23.3055x; 1.2336x over previous
import functools

import jax
import jax.numpy as jnp
from jax.experimental import pallas as pl

UP_RATIO = 10
K = 5
STD_FACTOR = 3.0

_SELF_TQ = 256
_MAIN_TQ = 256


def _coords(ref):
    return ref[0, :, 0:1], ref[0, :, 1:2], ref[0, :, 2:3]


def _rows(ref):
    return ref[0, 0:1, :], ref[0, 1:2, :], ref[0, 2:3, :]


def _bf16(x):
    return x.astype(jnp.bfloat16).astype(jnp.float32)


def _sel_and_exact_d2(q, p):
    qx, qy, qz = q
    px, py, pz = p
    q2 = qx * qx + qy * qy + qz * qz
    p2 = px * px + py * py + pz * pz
    qp2 = (_bf16(qx) * (-2.0 * _bf16(px)) + _bf16(qy) * (-2.0 * _bf16(py))
           + _bf16(qz) * (-2.0 * _bf16(pz)))
    d2_sel = (q2 + p2) + qp2
    dx = qx - px
    dy = qy - py
    dz = qz - pz
    d2_exact = dx * dx + dy * dy + dz * dz
    return d2_sel, d2_exact


def _self_knn_body(q_ref, pt_ref, out_ref, *, n):
    tq = q_ref.shape[1]
    q = _coords(q_ref)
    p = _rows(pt_ref)
    d2_sel, d2_exact = _sel_and_exact_d2(q, p)
    for rank in range(2):
        m = jnp.min(d2_sel, axis=1, keepdims=True)
        onehot = d2_sel <= m
        if rank == 1:
            out_ref[0] = jnp.sum(jnp.where(onehot, d2_exact, 0.0),
                                 axis=1, keepdims=True)
        else:
            d2_sel = jnp.where(onehot, jnp.inf, d2_sel)


def _main_body(q_ref, ptt_ref, pts_ref, pmh_ref, pml_ref, out_ref, *, n):
    tq = q_ref.shape[1]
    q = _coords(q_ref)

    def top5(pt_ref):
        d2_sel, d2_exact = _sel_and_exact_d2(q, _rows(pt_ref))
        dists = []
        hots = []
        for k in range(K):
            m = jnp.min(d2_sel, axis=1, keepdims=True)
            onehot = d2_sel <= m
            dists.append(jnp.sum(jnp.where(onehot, d2_exact, 0.0),
                                 axis=1, keepdims=True))
            hots.append(onehot)
            if k < K - 1:
                d2_sel = jnp.where(onehot, jnp.inf, d2_sel)
        return dists, hots

    mt, ht = top5(ptt_ref)
    inv = [1.0 / (m + 1e-8) for m in mt]
    norm = inv[0] + inv[1] + inv[2] + inv[3] + inv[4]
    rnorm = 1.0 / norm
    w = [inv[k] * rnorm for k in range(K)]

    udf_t = jnp.zeros((tq, 1), jnp.float32)
    u_t = jnp.zeros((tq, n), jnp.float32)
    for k in range(K):
        udf_t = udf_t + jnp.sqrt(mt[k] + 1e-10) * inv[k]
        u_t = u_t + jnp.where(ht[k], w[k], 0.0)

    ms, hs = top5(pts_ref)
    udf_s = jnp.zeros((tq, 1), jnp.float32)
    u_s = jnp.zeros((tq, n), jnp.float32)
    for k in range(K):
        udf_s = udf_s + jnp.sqrt(ms[k] + 1e-10) * inv[k]
        u_s = u_s - jnp.where(hs[k], w[k], 0.0)

    u = jnp.concatenate([u_t, u_s], axis=1).astype(jnp.bfloat16)
    wpd = (jnp.dot(u, pmh_ref[0], preferred_element_type=jnp.float32)
           + jnp.dot(u, pml_ref[0], preferred_element_type=jnp.float32))
    gd = jnp.sum(jnp.abs(wpd), axis=1, keepdims=True)
    err = jnp.abs(udf_t - udf_s) * rnorm + gd
    out_ref[0] = err


@jax.jit
def kernel(src, tgt):
    b, n, _ = tgt.shape
    nq = n * UP_RATIO + src.shape[1]

    tgt_t = jnp.swapaxes(tgt, 1, 2)
    src_t = jnp.swapaxes(src, 1, 2)

    self_d2 = pl.pallas_call(
        functools.partial(_self_knn_body, n=n),
        grid=(b, n // _SELF_TQ),
        in_specs=[
            pl.BlockSpec((1, _SELF_TQ, 3), lambda i, j: (i, j, 0)),
            pl.BlockSpec((1, 3, n), lambda i, j: (i, 0, 0)),
        ],
        out_specs=pl.BlockSpec((1, _SELF_TQ, 1), lambda i, j: (i, j, 0)),
        out_shape=jax.ShapeDtypeStruct((b, n, 1), jnp.float32),
    )(tgt, tgt_t)

    std = jnp.sqrt(self_d2 + 1e-10) * STD_FACTOR
    noise = jax.random.normal(
        jax.random.key(42), (b, n, UP_RATIO, 3), dtype=jnp.float32
    ) * std[..., None]
    query = (tgt[:, :, None, :] + noise).reshape(b, -1, 3)
    query = jnp.concatenate([query, src], axis=1)

    pad = jnp.zeros((b, n, 5), jnp.float32)
    tgt_pad = jnp.concatenate([tgt, pad], axis=2)
    src_pad = jnp.concatenate([src, pad], axis=2)
    pm = jnp.concatenate([tgt_pad, src_pad], axis=1)
    pm_hi = pm.astype(jnp.bfloat16)
    pm_lo = (pm - pm_hi.astype(jnp.float32)).astype(jnp.bfloat16)

    err = pl.pallas_call(
        functools.partial(_main_body, n=n),
        grid=(b, nq // _MAIN_TQ),
        in_specs=[
            pl.BlockSpec((1, _MAIN_TQ, 3), lambda i, j: (i, j, 0)),
            pl.BlockSpec((1, 3, n), lambda i, j: (i, 0, 0)),
            pl.BlockSpec((1, 3, n), lambda i, j: (i, 0, 0)),
            pl.BlockSpec((1, 2 * n, 8), lambda i, j: (i, 0, 0)),
            pl.BlockSpec((1, 2 * n, 8), lambda i, j: (i, 0, 0)),
        ],
        out_specs=pl.BlockSpec((1, _MAIN_TQ, 1), lambda i, j: (i, j, 0)),
        out_shape=jax.ShapeDtypeStruct((b, nq, 1), jnp.float32),
    )(query, tgt_t, src_t, pm_hi, pm_lo)

    return jnp.sum(err) / b / nq


# speedup vs baseline: 23.8443x; 1.0231x over previous
import functools

import jax
import jax.numpy as jnp
from jax.experimental import pallas as pl

UP_RATIO = 10
K = 5
STD_FACTOR = 3.0

_SELF_TQ = 256
_MAIN_TQ = 512


def _coords(ref):
    return ref[0, :, 0:1], ref[0, :, 1:2], ref[0, :, 2:3]


def _rows(ref):
    return ref[0, 0:1, :], ref[0, 1:2, :], ref[0, 2:3, :]


def _bf16(x):
    return x.astype(jnp.bfloat16).astype(jnp.float32)


def _sel_and_exact_d2(q, p):
    qx, qy, qz = q
    px, py, pz = p
    q2 = qx * qx + qy * qy + qz * qz
    p2 = px * px + py * py + pz * pz
    qp2 = (_bf16(qx) * (-2.0 * _bf16(px)) + _bf16(qy) * (-2.0 * _bf16(py))
           + _bf16(qz) * (-2.0 * _bf16(pz)))
    d2_sel = (q2 + p2) + qp2
    dx = qx - px
    dy = qy - py
    dz = qz - pz
    d2_exact = dx * dx + dy * dy + dz * dz
    return d2_sel, d2_exact


def _self_knn_body(q_ref, pt_ref, out_ref, *, n):
    tq = q_ref.shape[1]
    q = _coords(q_ref)
    p = _rows(pt_ref)
    d2_sel, d2_exact = _sel_and_exact_d2(q, p)
    for rank in range(2):
        m = jnp.min(d2_sel, axis=1, keepdims=True)
        onehot = d2_sel <= m
        if rank == 1:
            out_ref[0] = jnp.sum(jnp.where(onehot, d2_exact, 0.0),
                                 axis=1, keepdims=True)
        else:
            d2_sel = jnp.where(onehot, jnp.inf, d2_sel)


def _main_body(q_ref, ptt_ref, pts_ref, pmh_ref, pml_ref, out_ref, *, n):
    tq = q_ref.shape[1]
    q = _coords(q_ref)

    def top5(pt_ref):
        d2_sel, d2_exact = _sel_and_exact_d2(q, _rows(pt_ref))
        dists = []
        hots = []
        for k in range(K):
            m = jnp.min(d2_sel, axis=1, keepdims=True)
            onehot = d2_sel <= m
            dists.append(jnp.sum(jnp.where(onehot, d2_exact, 0.0),
                                 axis=1, keepdims=True))
            hots.append(onehot)
            if k < K - 1:
                d2_sel = jnp.where(onehot, jnp.inf, d2_sel)
        return dists, hots

    mt, ht = top5(ptt_ref)
    inv = [1.0 / (m + 1e-8) for m in mt]
    norm = inv[0] + inv[1] + inv[2] + inv[3] + inv[4]
    rnorm = 1.0 / norm
    w = [inv[k] * rnorm for k in range(K)]

    udf_t = jnp.zeros((tq, 1), jnp.float32)
    u_t = jnp.zeros((tq, n), jnp.float32)
    for k in range(K):
        udf_t = udf_t + jnp.sqrt(mt[k] + 1e-10) * inv[k]
        u_t = u_t + jnp.where(ht[k], w[k], 0.0)

    ms, hs = top5(pts_ref)
    udf_s = jnp.zeros((tq, 1), jnp.float32)
    u_s = jnp.zeros((tq, n), jnp.float32)
    for k in range(K):
        udf_s = udf_s + jnp.sqrt(ms[k] + 1e-10) * inv[k]
        u_s = u_s - jnp.where(hs[k], w[k], 0.0)

    u = jnp.concatenate([u_t, u_s], axis=1).astype(jnp.bfloat16)
    wpd = (jnp.dot(u, pmh_ref[0], preferred_element_type=jnp.float32)
           + jnp.dot(u, pml_ref[0], preferred_element_type=jnp.float32))
    gd = jnp.sum(jnp.abs(wpd), axis=1, keepdims=True)
    err = jnp.abs(udf_t - udf_s) * rnorm + gd
    out_ref[0] = err


@jax.jit
def kernel(src, tgt):
    b, n, _ = tgt.shape
    nq = n * UP_RATIO + src.shape[1]

    tgt_t = jnp.swapaxes(tgt, 1, 2)
    src_t = jnp.swapaxes(src, 1, 2)

    self_d2 = pl.pallas_call(
        functools.partial(_self_knn_body, n=n),
        grid=(b, n // _SELF_TQ),
        in_specs=[
            pl.BlockSpec((1, _SELF_TQ, 3), lambda i, j: (i, j, 0)),
            pl.BlockSpec((1, 3, n), lambda i, j: (i, 0, 0)),
        ],
        out_specs=pl.BlockSpec((1, _SELF_TQ, 1), lambda i, j: (i, j, 0)),
        out_shape=jax.ShapeDtypeStruct((b, n, 1), jnp.float32),
    )(tgt, tgt_t)

    std = jnp.sqrt(self_d2 + 1e-10) * STD_FACTOR
    noise = jax.random.normal(
        jax.random.key(42), (b, n, UP_RATIO, 3), dtype=jnp.float32
    ) * std[..., None]
    query = (tgt[:, :, None, :] + noise).reshape(b, -1, 3)
    query = jnp.concatenate([query, src], axis=1)

    pad = jnp.zeros((b, n, 5), jnp.float32)
    tgt_pad = jnp.concatenate([tgt, pad], axis=2)
    src_pad = jnp.concatenate([src, pad], axis=2)
    pm = jnp.concatenate([tgt_pad, src_pad], axis=1)
    pm_hi = pm.astype(jnp.bfloat16)
    pm_lo = (pm - pm_hi.astype(jnp.float32)).astype(jnp.bfloat16)

    err = pl.pallas_call(
        functools.partial(_main_body, n=n),
        grid=(b, nq // _MAIN_TQ),
        in_specs=[
            pl.BlockSpec((1, _MAIN_TQ, 3), lambda i, j: (i, j, 0)),
            pl.BlockSpec((1, 3, n), lambda i, j: (i, 0, 0)),
            pl.BlockSpec((1, 3, n), lambda i, j: (i, 0, 0)),
            pl.BlockSpec((1, 2 * n, 8), lambda i, j: (i, 0, 0)),
            pl.BlockSpec((1, 2 * n, 8), lambda i, j: (i, 0, 0)),
        ],
        out_specs=pl.BlockSpec((1, _MAIN_TQ, 1), lambda i, j: (i, j, 0)),
        out_shape=jax.ShapeDtypeStruct((b, nq, 1), jnp.float32),
    )(query, tgt_t, src_t, pm_hi, pm_lo)

    return jnp.sum(err) / b / nq
